# baseline (speedup 1.0000x reference)
import jax
import jax.numpy as jnp
from jax import lax
from jax.experimental import pallas as pl
from jax.experimental.pallas import tpu as pltpu


_BH = 4


def _body(steps_ref, tab_ref, out_ref, buf_ref, sem_ref):
    steps = steps_ref[...]
    tab = tab_ref[...]
    t = tab.shape[0]
    b = steps.shape[1]
    sub = lax.broadcasted_iota(jnp.int32, (t, b), 0)
    eq = sub == steps
    vals = jnp.sum(jnp.where(eq, tab, 0.0), axis=0, keepdims=True)
    buf_ref[...] = jnp.broadcast_to(vals[None, :, :], buf_ref.shape)

    c, h, w, _ = out_ref.shape
    nj = h // _BH
    copies = []
    for ci in range(c):
        for j in range(nj):
            cp = pltpu.make_async_copy(
                buf_ref,
                out_ref.at[ci, pl.ds(j * _BH, _BH)],
                sem_ref.at[ci * nj + j],
            )
            cp.start()
            copies.append(cp)
    for cp in copies:
        cp.wait()


def kernel(img, num_steps, alpha_bars):
    b, c, h, w = img.shape
    t_pad = 1024
    tab_col = jnp.zeros((t_pad, 1), jnp.float32).at[: alpha_bars.shape[0], 0].set(
        alpha_bars
    )
    steps_row = num_steps.reshape(1, b)
    ndma = c * (h // _BH)

    out_t = pl.pallas_call(
        _body,
        in_specs=[
            pl.BlockSpec(memory_space=pltpu.VMEM),
            pl.BlockSpec(memory_space=pltpu.VMEM),
        ],
        out_specs=pl.BlockSpec(memory_space=pl.ANY),
        out_shape=jax.ShapeDtypeStruct((c, h, w, b), jnp.float32),
        scratch_shapes=[
            pltpu.VMEM((_BH, w, b), jnp.float32),
            pltpu.SemaphoreType.DMA((ndma,)),
        ],
    )(steps_row, tab_col)
    return jnp.transpose(out_t, (3, 0, 1, 2))
